# Initial kernel scaffold; baseline (speedup 1.0000x reference)
import jax
import jax.numpy as jnp
from jax.experimental import pallas as pl


def kernel(h, embedding):
    raise NotImplementedError("write your pallas kernel here")


# speedup vs baseline: 1.4070x; 1.4070x over previous
import functools

import jax
import jax.numpy as jnp
from jax import lax
from jax.experimental import pallas as pl
from jax.experimental.pallas import tpu as pltpu
from jax.experimental.pallas import tpu_sc as plsc

VOCAB = 8192
D_EMBED = 32
N_TOKENS = 16384

ROW_TILE = 2048
N_ROW_TILES = N_TOKENS // ROW_TILE
V_CHUNK = 2048
N_V_CHUNKS = VOCAB // V_CHUNK
LOSS_SCALE = 1.0 / (N_TOKENS * D_EMBED)


def _bf16_rne(x):
    b = lax.bitcast_convert_type(x, jnp.uint32)
    b = (b + jnp.uint32(0x7FFF) + ((b >> 16) & jnp.uint32(1))) & jnp.uint32(0xFFFF0000)
    return lax.bitcast_convert_type(b, jnp.float32)


def _argmin_body(ht_ref, emb_ref, idx_ref, loss_ref):
    i = pl.program_id(0)
    hblk = ht_ref[...]
    hh = jnp.sum(hblk * hblk, axis=0, keepdims=True)
    hblk_b = hblk.astype(jnp.bfloat16)

    def chunk_min(c):
        emb_c = emb_ref[pl.ds(c * V_CHUNK, V_CHUNK), :]
        ee = jnp.sum(emb_c * emb_c, axis=1, keepdims=True)
        dot = jnp.dot(
            emb_c.astype(jnp.bfloat16),
            hblk_b,
            preferred_element_type=jnp.float32,
        )
        dist = (hh + ee) - 2.0 * dot
        lmin = jnp.min(dist, axis=0)
        cols = lax.broadcasted_iota(jnp.int32, dist.shape, 0)
        larg = jnp.min(jnp.where(dist == lmin[None, :], cols, V_CHUNK), axis=0)
        return lmin, larg + (c * V_CHUNK)

    def merge_f32(a, b):
        av, ai = a
        bv, bi = b
        upd = bv < av
        return jnp.where(upd, bv, av), jnp.where(upd, bi, ai)

    win_a = merge_f32(chunk_min(0), chunk_min(1))
    win_b = merge_f32(chunk_min(2), chunk_min(3))
    carry_q = _bf16_rne(win_a[0])
    upd = win_b[0] < carry_q
    best_idx = jnp.where(upd, win_b[1], win_a[1])
    best_f32 = jnp.where(upd, win_b[0], win_a[0])

    idx_ref[0, 0, :] = best_idx
    tile_loss = jnp.sum(best_f32) * LOSS_SCALE

    @pl.when(i == 0)
    def _init():
        loss_ref[0, 0] = tile_loss

    @pl.when(i > 0)
    def _acc():
        loss_ref[0, 0] += tile_loss


def _distance_argmin(h_t, embedding):
    return pl.pallas_call(
        _argmin_body,
        grid=(N_ROW_TILES,),
        in_specs=[
            pl.BlockSpec((D_EMBED, ROW_TILE), lambda i: (0, i)),
            pl.BlockSpec((VOCAB, D_EMBED), lambda i: (0, 0)),
        ],
        out_specs=[
            pl.BlockSpec((1, 1, ROW_TILE), lambda i: (i, 0, 0)),
            pl.BlockSpec(memory_space=pltpu.SMEM),
        ],
        out_shape=[
            jax.ShapeDtypeStruct((N_ROW_TILES, 1, ROW_TILE), jnp.int32),
            jax.ShapeDtypeStruct((1, 1), jnp.float32),
        ],
    )(h_t, embedding)


_NC, _NS = 2, 16
_NW = _NC * _NS
_ROWS_PER_W = N_TOKENS // _NW
_IDX_CHUNK = 128
_N_IDX_CHUNKS = _ROWS_PER_W // _IDX_CHUNK


def _sc_gather(idx3, embedding):
    mesh = plsc.VectorSubcoreMesh(core_axis_name="c", subcore_axis_name="s")

    @functools.partial(
        pl.kernel,
        mesh=mesh,
        out_type=jax.ShapeDtypeStruct((N_TOKENS, D_EMBED), jnp.float32),
        scratch_types=[
            pltpu.VMEM((_N_IDX_CHUNKS, _IDX_CHUNK), jnp.int32),
            pltpu.VMEM((_ROWS_PER_W, D_EMBED), jnp.float32),
            pltpu.SemaphoreType.DMA,
        ],
        compiler_params=pltpu.CompilerParams(use_tc_tiling_on_sc=False),
    )
    def gather_k(idx_hbm, table_hbm, out_hbm, idx_v, rows_v, sem):
        wid = lax.axis_index("s") * _NC + lax.axis_index("c")
        base = wid * _ROWS_PER_W
        pltpu.sync_copy(idx_hbm.at[wid], idx_v)
        cps = [
            pltpu.async_copy(
                table_hbm.at[idx_v.at[j]],
                rows_v.at[pl.ds(j * _IDX_CHUNK, _IDX_CHUNK)],
                sem,
            )
            for j in range(_N_IDX_CHUNKS)
        ]
        for cp in cps:
            cp.wait()
        pltpu.sync_copy(rows_v, out_hbm.at[pl.ds(base, _ROWS_PER_W)])

    return gather_k(idx3, embedding)


def kernel(h, embedding):
    B, C, H, W = h.shape
    h = h.astype(jnp.float32)
    h_t = jnp.transpose(h, (1, 0, 2, 3)).reshape(C, B * H * W)

    idx_tiles, loss = _distance_argmin(h_t, embedding)
    indices_flat = idx_tiles.reshape(B * H * W)

    idx3 = indices_flat.reshape(_NW, _N_IDX_CHUNKS, _IDX_CHUNK)
    z_q_flat = _sc_gather(idx3, embedding)

    z_q = jnp.transpose(z_q_flat.reshape(B, H, W, C), (0, 3, 1, 2))
    indices = indices_flat.reshape(B, H, W)
    commit_loss = loss[0, 0]
    z_q_st = h + (z_q - h)
    return (z_q_st, indices, commit_loss)


# speedup vs baseline: 1.4627x; 1.0396x over previous
import functools

import jax
import jax.numpy as jnp
from jax import lax
from jax.experimental import pallas as pl
from jax.experimental.pallas import tpu as pltpu
from jax.experimental.pallas import tpu_sc as plsc

VOCAB = 8192
D_EMBED = 32
N_TOKENS = 16384

ROW_TILE = 1024
N_ROW_TILES = N_TOKENS // ROW_TILE
V_CHUNK = 2048
N_V_CHUNKS = VOCAB // V_CHUNK
LOSS_SCALE = 1.0 / (N_TOKENS * D_EMBED)


def _bf16_rne(x):
    b = lax.bitcast_convert_type(x, jnp.uint32)
    b = (b + jnp.uint32(0x7FFF) + ((b >> 16) & jnp.uint32(1))) & jnp.uint32(0xFFFF0000)
    return lax.bitcast_convert_type(b, jnp.float32)


def _argmin_body(ht_ref, emb_ref, idx_ref, loss_ref):
    i = pl.program_id(0)
    hblk = ht_ref[0]
    hh = jnp.sum(hblk * hblk, axis=0, keepdims=True)
    hblk_b = (hblk + hblk).astype(jnp.bfloat16)
    cols = lax.broadcasted_iota(jnp.int32, (V_CHUNK, ROW_TILE), 0)

    def chunk_min(c):
        emb_c = emb_ref[pl.ds(c * V_CHUNK, V_CHUNK), :]
        ee = jnp.sum(emb_c * emb_c, axis=1, keepdims=True)
        dot2 = jnp.dot(
            emb_c.astype(jnp.bfloat16),
            hblk_b,
            preferred_element_type=jnp.float32,
        )
        dist = (hh + ee) - dot2
        lmin = jnp.min(dist, axis=0)
        larg = jnp.min(jnp.where(dist == lmin[None, :], cols, V_CHUNK), axis=0)
        return lmin, larg + (c * V_CHUNK)

    def merge_f32(a, b):
        av, ai = a
        bv, bi = b
        upd = bv < av
        return jnp.where(upd, bv, av), jnp.where(upd, bi, ai)

    win_a = merge_f32(chunk_min(0), chunk_min(1))
    win_b = merge_f32(chunk_min(2), chunk_min(3))
    carry_q = _bf16_rne(win_a[0])
    upd = win_b[0] < carry_q
    best_idx = jnp.where(upd, win_b[1], win_a[1])
    best_f32 = jnp.where(upd, win_b[0], win_a[0])

    idx_ref[0, 0, :] = best_idx
    tile_loss = jnp.sum(best_f32) * LOSS_SCALE

    @pl.when(i == 0)
    def _init():
        loss_ref[0, 0] = tile_loss

    @pl.when(i > 0)
    def _acc():
        loss_ref[0, 0] += tile_loss


def _distance_argmin(h3, embedding):
    return pl.pallas_call(
        _argmin_body,
        grid=(N_ROW_TILES,),
        in_specs=[
            pl.BlockSpec((1, D_EMBED, ROW_TILE), lambda i: (i, 0, 0)),
            pl.BlockSpec((VOCAB, D_EMBED), lambda i: (0, 0)),
        ],
        out_specs=[
            pl.BlockSpec((1, 1, ROW_TILE), lambda i: (i, 0, 0)),
            pl.BlockSpec(memory_space=pltpu.SMEM),
        ],
        out_shape=[
            jax.ShapeDtypeStruct((N_ROW_TILES, 1, ROW_TILE), jnp.int32),
            jax.ShapeDtypeStruct((1, 1), jnp.float32),
        ],
    )(h3, embedding)


_NC, _NS = 2, 16
_NW = _NC * _NS
_ROWS_PER_W = N_TOKENS // _NW
_IDX_CHUNK = 128
_N_IDX_CHUNKS = _ROWS_PER_W // _IDX_CHUNK


def _sc_gather(idx3, embedding):
    mesh = plsc.VectorSubcoreMesh(core_axis_name="c", subcore_axis_name="s")

    @functools.partial(
        pl.kernel,
        mesh=mesh,
        out_type=jax.ShapeDtypeStruct((N_TOKENS, D_EMBED), jnp.float32),
        scratch_types=[
            pltpu.VMEM((_N_IDX_CHUNKS, _IDX_CHUNK), jnp.int32),
            pltpu.VMEM((_ROWS_PER_W, D_EMBED), jnp.float32),
            pltpu.SemaphoreType.DMA,
        ],
        compiler_params=pltpu.CompilerParams(use_tc_tiling_on_sc=False),
    )
    def gather_k(idx_hbm, table_hbm, out_hbm, idx_v, rows_v, sem):
        wid = lax.axis_index("s") * _NC + lax.axis_index("c")
        base = wid * _ROWS_PER_W
        pltpu.sync_copy(idx_hbm.at[wid], idx_v)
        cps = [
            pltpu.async_copy(
                table_hbm.at[idx_v.at[j]],
                rows_v.at[pl.ds(j * _IDX_CHUNK, _IDX_CHUNK)],
                sem,
            )
            for j in range(_N_IDX_CHUNKS)
        ]
        for cp in cps:
            cp.wait()
        pltpu.sync_copy(rows_v, out_hbm.at[pl.ds(base, _ROWS_PER_W)])

    return gather_k(idx3, embedding)


def kernel(h, embedding):
    B, C, H, W = h.shape
    h = h.astype(jnp.float32)
    h3 = h.reshape(B, C, H * W)

    idx_tiles, loss = _distance_argmin(h3, embedding)
    indices_flat = idx_tiles.reshape(B * H * W)

    idx3 = indices_flat.reshape(_NW, _N_IDX_CHUNKS, _IDX_CHUNK)
    z_q_flat = _sc_gather(idx3, embedding)

    z_q = jnp.transpose(z_q_flat.reshape(B, H, W, C), (0, 3, 1, 2))
    indices = indices_flat.reshape(B, H, W)
    commit_loss = loss[0, 0]
    z_q_st = h + (z_q - h)
    return (z_q_st, indices, commit_loss)
